# baseline (speedup 1.0000x reference)
import functools

import jax
import jax.numpy as jnp
import numpy as np
from jax import lax
from jax.experimental import pallas as pl
from jax.experimental.pallas import tpu as pltpu
from jax.experimental.pallas import tpu_sc as plsc

_FIELD_DIMS = [100000] * 26
_F = len(_FIELD_DIMS)
_E = 16
_B = 16384
_N = _B * _F
_NUM_EMB = sum(_FIELD_DIMS)
_OFFSETS_NP = np.concatenate(([0], np.cumsum(_FIELD_DIMS[:-1]))).astype(np.int32)

_NC, _NS, _L = 2, 16, 16
_NW = _NC * _NS

_PER_W = _N // _NW
_CHUNK = 3328
_NCHUNK = _PER_W // _CHUNK
_OFFS_TILED_NP = np.tile(_OFFSETS_NP, _CHUNK // _F)

_SLAB_COLS = 1024
_TILES_FULL = _NUM_EMB // 128
_NSLAB_FULL = _TILES_FULL // 8
_SLABS_PER_W = (_NSLAB_FULL + _NW - 1) // _NW
_TAIL_COLS = _NUM_EMB - _TILES_FULL * 128
_STAGE = _SLAB_COLS * _E


def _make_sc_transpose():
    mesh = plsc.VectorSubcoreMesh(core_axis_name="c", subcore_axis_name="s")

    @functools.partial(
        pl.kernel,
        mesh=mesh,
        out_type=jax.ShapeDtypeStruct((_NUM_EMB * _E,), jnp.float32),
        compiler_params=pltpu.CompilerParams(
            use_tc_tiling_on_sc=True, needs_layout_passes=False),
        scratch_types=[
            pltpu.VMEM((8, _SLAB_COLS), jnp.float32),
            pltpu.VMEM((8, _SLAB_COLS), jnp.float32),
            pltpu.VMEM((8, _SLAB_COLS), jnp.float32),
            pltpu.VMEM((8, _SLAB_COLS), jnp.float32),
            pltpu.VMEM((_STAGE,), jnp.float32),
            pltpu.VMEM((_STAGE,), jnp.float32),
            pltpu.SemaphoreType.DMA,
            pltpu.SemaphoreType.DMA,
            pltpu.SemaphoreType.DMA,
            pltpu.SemaphoreType.DMA,
        ],
    )
    def tr_kernel(t4_hbm, tail_hbm, out_hbm, a0, b0, a1, b1, st0, st1,
                  gs0, gs1, ss0, ss1):
        wid = lax.axis_index("s") * _NC + lax.axis_index("c")
        iota16 = lax.iota(jnp.int32, 16)
        lane16 = iota16 * _E
        bufs = [(a0, b0), (a1, b1)]
        stages = [st0, st1]
        gsems = [gs0, gs1]
        ssems = [ss0, ss1]

        def slab_id(k):
            return k * _NW + wid

        def issue_in(k, par):
            s = slab_id(k)

            @pl.when(s < _NSLAB_FULL)
            def _():
                c0 = s * _SLAB_COLS
                pltpu.async_copy(
                    t4_hbm.at[0, :, pl.ds(c0, _SLAB_COLS)], bufs[par][0],
                    gsems[par])
                pltpu.async_copy(
                    t4_hbm.at[1, :, pl.ds(c0, _SLAB_COLS)], bufs[par][1],
                    gsems[par])

        def wait_in(k, par):
            s = slab_id(k)

            @pl.when(s < _NSLAB_FULL)
            def _():
                c0 = s * _SLAB_COLS
                pltpu.make_async_copy(
                    t4_hbm.at[0, :, pl.ds(c0, _SLAB_COLS)], bufs[par][0],
                    gsems[par]).wait()
                pltpu.make_async_copy(
                    t4_hbm.at[1, :, pl.ds(c0, _SLAB_COLS)], bufs[par][1],
                    gsems[par]).wait()

        def wait_out(k, par):
            s = slab_id(k)

            @pl.when(s < _NSLAB_FULL)
            def _():
                pltpu.make_async_copy(
                    stages[par],
                    out_hbm.at[pl.ds(s * _STAGE, _STAGE)],
                    ssems[par]).wait()

        def process(k, par):
            s = slab_id(k)

            @pl.when(s < _NSLAB_FULL)
            def _():
                ba, bb = bufs[par]
                stage = stages[par]

                def pair_body(p, carry):
                    base = p * 128
                    sbase = p * 2048
                    for c in range(8):
                        src = base + c * 16
                        dst = sbase + c * 256 + lane16
                        for e in range(8):
                            plsc.store_scatter(
                                stage, [dst + e],
                                ba[e, pl.ds(src, 16)])
                            plsc.store_scatter(
                                stage, [dst + (e + 8)],
                                bb[e, pl.ds(src, 16)])
                    return carry

                lax.fori_loop(0, _SLAB_COLS // 128, pair_body, 0)
                pltpu.async_copy(
                    stage, out_hbm.at[pl.ds(s * _STAGE, _STAGE)], ssems[par])

        issue_in(0, 0)

        def step(k, par):
            wait_in(k, par)

            @pl.when(k >= 2)
            def _():
                wait_out(k - 2, par)

            process(k, par)

        def pipe_body(i, carry):
            k0 = i * 2
            issue_in(k0 + 1, 1)
            step(k0, 0)
            issue_in(k0 + 2, 0)
            step(k0 + 1, 1)
            return carry

        lax.fori_loop(0, _SLABS_PER_W // 2, pipe_body, 0)
        wait_out(_SLABS_PER_W - 2, 0)
        wait_out(_SLABS_PER_W - 1, 1)

        @pl.when(wid == 0)
        def _():
            n_tail = _TAIL_COLS * _E
            pltpu.sync_copy(tail_hbm, st1.at[pl.ds(0, n_tail)])
            pltpu.sync_copy(
                st1.at[pl.ds(0, n_tail)],
                out_hbm.at[pl.ds(_TILES_FULL * 128 * _E, n_tail)])

    return tr_kernel


def _make_outfmt():
    mesh = plsc.VectorSubcoreMesh(core_axis_name="c", subcore_axis_name="s")
    blocks_per_w = (_B // 128) // _NW

    @functools.partial(
        pl.kernel,
        mesh=mesh,
        out_type=jax.ShapeDtypeStruct((_F, _E, _B), jnp.float32),
        compiler_params=pltpu.CompilerParams(
            use_tc_tiling_on_sc=True, needs_layout_passes=False),
        scratch_types=[
            pltpu.VMEM((_CHUNK * _E,), jnp.float32),
            pltpu.VMEM((_CHUNK * _E,), jnp.float32),
            pltpu.VMEM((_E, 128), jnp.float32),
            pltpu.VMEM((_E, 128), jnp.float32),
            pltpu.SemaphoreType.DMA,
            pltpu.SemaphoreType.DMA,
            pltpu.SemaphoreType.DMA,
            pltpu.SemaphoreType.DMA,
        ],
    )
    def outfmt_kernel(rows_hbm, out_hbm, sl0, sl1, st0, st1,
                      gs0, gs1, ss0, ss1):
        wid = lax.axis_index("s") * _NC + lax.axis_index("c")
        iota16 = lax.iota(jnp.int32, 16)
        slabs = [sl0, sl1]
        stages = [st0, st1]
        gsems = [gs0, gs1]
        ssems = [ss0, ss1]

        def blk_base(k):
            return (wid * blocks_per_w + k) * 128

        def issue_slab(k, par):
            if k >= blocks_per_w:
                return
            b0 = blk_base(k)
            pltpu.async_copy(
                rows_hbm.at[pl.ds(b0 * _F * _E, _CHUNK * _E)],
                slabs[par], gsems[par])

        def wait_slab(k, par):
            b0 = blk_base(k)
            pltpu.make_async_copy(
                rows_hbm.at[pl.ds(b0 * _F * _E, _CHUNK * _E)],
                slabs[par], gsems[par]).wait()

        issue_slab(0, 0)
        stage = st0
        for k in range(blocks_per_w):
            par = k % 2
            issue_slab(k + 1, 1 - par)
            wait_slab(k, par)
            slab = slabs[par]
            b0 = blk_base(k)

            def f_body(f, carry, _slab=slab, _b0=b0):
                @pl.when(f >= 1)
                def _():
                    pltpu.make_async_copy(
                        stage, out_hbm.at[f - 1, :, pl.ds(_b0, 128)],
                        ss0).wait()

                def row_body(blk, carry2):
                    for j in range(8):
                        bl = blk * 8 + j
                        v = _slab[pl.ds((bl * _F + f) * _E, _E)]
                        bl_vec = jnp.full((16,), bl, jnp.int32)
                        plsc.store_scatter(stage, [iota16, bl_vec], v)
                    return carry2

                lax.fori_loop(0, 128 // 8, row_body, 0)
                pltpu.async_copy(
                    stage, out_hbm.at[f, :, pl.ds(_b0, 128)], ss0)
                return carry

            lax.fori_loop(0, _F, f_body, 0)
            pltpu.make_async_copy(
                stage, out_hbm.at[_F - 1, :, pl.ds(b0, 128)], ss0).wait()

    return outfmt_kernel


def _make_gather():
    mesh = plsc.VectorSubcoreMesh(core_axis_name="c", subcore_axis_name="s")

    @functools.partial(
        pl.kernel,
        mesh=mesh,
        out_type=jax.ShapeDtypeStruct((_N, _E), jnp.float32),
        compiler_params=pltpu.CompilerParams(use_tc_tiling_on_sc=False),
        scratch_types=[
            pltpu.VMEM((_CHUNK,), jnp.int32),
            pltpu.VMEM((_CHUNK,), jnp.int32),
            pltpu.VMEM((_CHUNK,), jnp.int32),
            pltpu.VMEM((_CHUNK, _E), jnp.float32),
            pltpu.VMEM((_CHUNK, _E), jnp.float32),
            pltpu.SemaphoreType.DMA,
            pltpu.SemaphoreType.DMA,
            pltpu.SemaphoreType.DMA,
            pltpu.SemaphoreType.DMA,
        ],
    )
    def gather_kernel(x_hbm, offs_hbm, table_hbm, out_hbm,
                      off_v, idx_a, idx_b, rows_a, rows_b,
                      gsem_a, gsem_b, ssem_a, ssem_b):
        wid = lax.axis_index("s") * _NC + lax.axis_index("c")
        base = wid * _PER_W
        pltpu.sync_copy(offs_hbm, off_v)
        idx = [idx_a, idx_b]
        rows = [rows_a, rows_b]
        gsem = [gsem_a, gsem_b]
        ssem = [ssem_a, ssem_b]
        gh = [None, None]
        sh = [None, None]
        for c in range(_NCHUNK):
            b = c % 2
            if c >= 2:
                sh[b].wait()
            start = base + c * _CHUNK
            pltpu.sync_copy(x_hbm.at[pl.ds(start, _CHUNK)], idx[b])

            def add_body(j, carry, _ib=idx[b]):
                sl = pl.ds(j * _L, _L)
                _ib[sl] = _ib[sl] + off_v[sl]
                return carry

            lax.fori_loop(0, _CHUNK // _L, add_body, 0)
            gh[b] = pltpu.async_copy(table_hbm.at[idx[b]], rows[b], gsem[b])
            if c >= 1:
                pb = 1 - b
                gh[pb].wait()
                sh[pb] = pltpu.async_copy(
                    rows[pb],
                    out_hbm.at[pl.ds(base + (c - 1) * _CHUNK, _CHUNK)],
                    ssem[pb],
                )
        last = (_NCHUNK - 1) % 2
        gh[last].wait()
        sh[last] = pltpu.async_copy(
            rows[last],
            out_hbm.at[pl.ds(base + (_NCHUNK - 1) * _CHUNK, _CHUNK)],
            ssem[last],
        )
        sh[0].wait()
        sh[1].wait()

    return gather_kernel


_SC_TRANSPOSE = _make_sc_transpose()
_GATHER = _make_gather()
_OUTFMT = _make_outfmt()


def kernel(x, table):
    x_flat = x.reshape(_N)
    offs = jnp.asarray(_OFFS_TILED_NP)
    t4 = table.T.reshape(2, 8, _NUM_EMB)
    tail_flat = table[_TILES_FULL * 128:, :].reshape(_TAIL_COLS * _E)
    table_lin = _SC_TRANSPOSE(t4, tail_flat).reshape(_NUM_EMB, _E)
    rows = _GATHER(x_flat, offs, table_lin)
    out3 = _OUTFMT(rows.reshape(_N * _E))
    return out3.transpose(2, 0, 1)


# speedup vs baseline: 1.0242x; 1.0242x over previous
import functools

import jax
import jax.numpy as jnp
import numpy as np
from jax import lax
from jax.experimental import pallas as pl
from jax.experimental.pallas import tpu as pltpu
from jax.experimental.pallas import tpu_sc as plsc

_FIELD_DIMS = [100000] * 26
_F = len(_FIELD_DIMS)
_E = 16
_B = 16384
_N = _B * _F
_NUM_EMB = sum(_FIELD_DIMS)
_OFFSETS_NP = np.concatenate(([0], np.cumsum(_FIELD_DIMS[:-1]))).astype(np.int32)

_NC, _NS, _L = 2, 16, 16
_NW = _NC * _NS

_PER_W = _N // _NW
_CHUNK = 3328
_NCHUNK = _PER_W // _CHUNK
_OFFS_TILED_NP = np.tile(_OFFSETS_NP, _CHUNK // _F)

_SLAB_COLS = 1024
_TILES_FULL = _NUM_EMB // 128
_NSLAB_FULL = _TILES_FULL // 8
_SLABS_PER_W = (_NSLAB_FULL + _NW - 1) // _NW
_TAIL_COLS = _NUM_EMB - _TILES_FULL * 128
_STAGE = _SLAB_COLS * _E


def _make_sc_transpose():
    mesh = plsc.VectorSubcoreMesh(core_axis_name="c", subcore_axis_name="s")

    @functools.partial(
        pl.kernel,
        mesh=mesh,
        out_type=jax.ShapeDtypeStruct((_NUM_EMB * _E,), jnp.float32),
        compiler_params=pltpu.CompilerParams(
            use_tc_tiling_on_sc=True, needs_layout_passes=False),
        scratch_types=[
            pltpu.VMEM((8, _SLAB_COLS), jnp.float32),
            pltpu.VMEM((8, _SLAB_COLS), jnp.float32),
            pltpu.VMEM((8, _SLAB_COLS), jnp.float32),
            pltpu.VMEM((8, _SLAB_COLS), jnp.float32),
            pltpu.VMEM((_STAGE,), jnp.float32),
            pltpu.VMEM((_STAGE,), jnp.float32),
            pltpu.SemaphoreType.DMA,
            pltpu.SemaphoreType.DMA,
            pltpu.SemaphoreType.DMA,
            pltpu.SemaphoreType.DMA,
        ],
    )
    def tr_kernel(t4_hbm, tail_hbm, out_hbm, a0, b0, a1, b1, st0, st1,
                  gs0, gs1, ss0, ss1):
        wid = lax.axis_index("s") * _NC + lax.axis_index("c")
        iota16 = lax.iota(jnp.int32, 16)
        lane16 = iota16 * _E
        bufs = [(a0, b0), (a1, b1)]
        stages = [st0, st1]
        gsems = [gs0, gs1]
        ssems = [ss0, ss1]

        def slab_id(k):
            return k * _NW + wid

        def issue_in(k, par):
            s = slab_id(k)

            @pl.when(s < _NSLAB_FULL)
            def _():
                c0 = s * _SLAB_COLS
                pltpu.async_copy(
                    t4_hbm.at[0, :, pl.ds(c0, _SLAB_COLS)], bufs[par][0],
                    gsems[par])
                pltpu.async_copy(
                    t4_hbm.at[1, :, pl.ds(c0, _SLAB_COLS)], bufs[par][1],
                    gsems[par])

        def wait_in(k, par):
            s = slab_id(k)

            @pl.when(s < _NSLAB_FULL)
            def _():
                c0 = s * _SLAB_COLS
                pltpu.make_async_copy(
                    t4_hbm.at[0, :, pl.ds(c0, _SLAB_COLS)], bufs[par][0],
                    gsems[par]).wait()
                pltpu.make_async_copy(
                    t4_hbm.at[1, :, pl.ds(c0, _SLAB_COLS)], bufs[par][1],
                    gsems[par]).wait()

        def wait_out(k, par):
            s = slab_id(k)

            @pl.when(s < _NSLAB_FULL)
            def _():
                pltpu.make_async_copy(
                    stages[par],
                    out_hbm.at[pl.ds(s * _STAGE, _STAGE)],
                    ssems[par]).wait()

        def process(k, par):
            s = slab_id(k)

            @pl.when(s < _NSLAB_FULL)
            def _():
                ba, bb = bufs[par]
                stage = stages[par]

                def pair_body(p, carry):
                    base = p * 128
                    sbase = p * 2048
                    for c in range(8):
                        src = base + c * 16
                        dst = sbase + c * 256 + lane16
                        for e in range(8):
                            plsc.store_scatter(
                                stage, [dst + e],
                                ba[e, pl.ds(src, 16)])
                            plsc.store_scatter(
                                stage, [dst + (e + 8)],
                                bb[e, pl.ds(src, 16)])
                    return carry

                lax.fori_loop(0, _SLAB_COLS // 128, pair_body, 0)
                pltpu.async_copy(
                    stage, out_hbm.at[pl.ds(s * _STAGE, _STAGE)], ssems[par])

        issue_in(0, 0)

        def step(k, par):
            wait_in(k, par)

            @pl.when(k >= 2)
            def _():
                wait_out(k - 2, par)

            process(k, par)

        def pipe_body(i, carry):
            k0 = i * 2
            issue_in(k0 + 1, 1)
            step(k0, 0)
            issue_in(k0 + 2, 0)
            step(k0 + 1, 1)
            return carry

        lax.fori_loop(0, _SLABS_PER_W // 2, pipe_body, 0)
        wait_out(_SLABS_PER_W - 2, 0)
        wait_out(_SLABS_PER_W - 1, 1)

        @pl.when(wid == 0)
        def _():
            n_tail = _TAIL_COLS * _E
            pltpu.sync_copy(tail_hbm, st1.at[pl.ds(0, n_tail)])
            pltpu.sync_copy(
                st1.at[pl.ds(0, n_tail)],
                out_hbm.at[pl.ds(_TILES_FULL * 128 * _E, n_tail)])

    return tr_kernel


def _make_outfmt():
    mesh = plsc.VectorSubcoreMesh(core_axis_name="c", subcore_axis_name="s")
    blocks_per_w = (_B // 128) // _NW

    @functools.partial(
        pl.kernel,
        mesh=mesh,
        out_type=jax.ShapeDtypeStruct((_F, _E, _B), jnp.float32),
        compiler_params=pltpu.CompilerParams(
            use_tc_tiling_on_sc=True, needs_layout_passes=False),
        scratch_types=[
            pltpu.VMEM((_CHUNK * _E,), jnp.float32),
            pltpu.VMEM((_CHUNK * _E,), jnp.float32),
            pltpu.VMEM((_E, 128), jnp.float32),
            pltpu.VMEM((_E, 128), jnp.float32),
            pltpu.SemaphoreType.DMA,
            pltpu.SemaphoreType.DMA,
            pltpu.SemaphoreType.DMA,
            pltpu.SemaphoreType.DMA,
        ],
    )
    def outfmt_kernel(rows_hbm, out_hbm, sl0, sl1, st0, st1,
                      gs0, gs1, ss0, ss1):
        wid = lax.axis_index("s") * _NC + lax.axis_index("c")
        iota16 = lax.iota(jnp.int32, 16)
        slabs = [sl0, sl1]
        stages = [st0, st1]
        gsems = [gs0, gs1]
        ssems = [ss0, ss1]

        def blk_base(k):
            return (wid * blocks_per_w + k) * 128

        def issue_slab(k, par):
            if k >= blocks_per_w:
                return
            b0 = blk_base(k)
            pltpu.async_copy(
                rows_hbm.at[pl.ds(b0 * _F * _E, _CHUNK * _E)],
                slabs[par], gsems[par])

        def wait_slab(k, par):
            b0 = blk_base(k)
            pltpu.make_async_copy(
                rows_hbm.at[pl.ds(b0 * _F * _E, _CHUNK * _E)],
                slabs[par], gsems[par]).wait()

        issue_slab(0, 0)
        for k in range(blocks_per_w):
            par = k % 2
            issue_slab(k + 1, 1 - par)
            wait_slab(k, par)
            slab = slabs[par]
            b0 = blk_base(k)

            def f_body(fi, carry, _slab=slab, _b0=b0):
                for sp in range(2):
                    f = fi * 2 + sp
                    stage = stages[sp]

                    @pl.when(fi >= 1)
                    def _(stage=stage, f=f):
                        pltpu.make_async_copy(
                            stage, out_hbm.at[f - 2, :, pl.ds(_b0, 128)],
                            ssems[sp]).wait()

                    def row_body(blk, carry2, stage=stage, f=f):
                        for j in range(8):
                            bl = blk * 8 + j
                            v = _slab[pl.ds((bl * _F + f) * _E, _E)]
                            bl_vec = jnp.full((16,), bl, jnp.int32)
                            plsc.store_scatter(stage, [iota16, bl_vec], v)
                        return carry2

                    lax.fori_loop(0, 128 // 8, row_body, 0)
                    pltpu.async_copy(
                        stage, out_hbm.at[f, :, pl.ds(_b0, 128)], ssems[sp])
                return carry

            lax.fori_loop(0, _F // 2, f_body, 0)
            for sp in range(2):
                pltpu.make_async_copy(
                    stages[sp], out_hbm.at[_F - 2 + sp, :, pl.ds(b0, 128)],
                    ssems[sp]).wait()

    return outfmt_kernel


def _make_gather():
    mesh = plsc.VectorSubcoreMesh(core_axis_name="c", subcore_axis_name="s")

    @functools.partial(
        pl.kernel,
        mesh=mesh,
        out_type=jax.ShapeDtypeStruct((_N, _E), jnp.float32),
        compiler_params=pltpu.CompilerParams(use_tc_tiling_on_sc=False),
        scratch_types=[
            pltpu.VMEM((_CHUNK,), jnp.int32),
            pltpu.VMEM((_CHUNK,), jnp.int32),
            pltpu.VMEM((_CHUNK,), jnp.int32),
            pltpu.VMEM((_CHUNK, _E), jnp.float32),
            pltpu.VMEM((_CHUNK, _E), jnp.float32),
            pltpu.SemaphoreType.DMA,
            pltpu.SemaphoreType.DMA,
            pltpu.SemaphoreType.DMA,
            pltpu.SemaphoreType.DMA,
        ],
    )
    def gather_kernel(x_hbm, offs_hbm, table_hbm, out_hbm,
                      off_v, idx_a, idx_b, rows_a, rows_b,
                      gsem_a, gsem_b, ssem_a, ssem_b):
        wid = lax.axis_index("s") * _NC + lax.axis_index("c")
        base = wid * _PER_W
        pltpu.sync_copy(offs_hbm, off_v)
        idx = [idx_a, idx_b]
        rows = [rows_a, rows_b]
        gsem = [gsem_a, gsem_b]
        ssem = [ssem_a, ssem_b]
        gh = [None, None]
        sh = [None, None]
        for c in range(_NCHUNK):
            b = c % 2
            if c >= 2:
                sh[b].wait()
            start = base + c * _CHUNK
            pltpu.sync_copy(x_hbm.at[pl.ds(start, _CHUNK)], idx[b])

            def add_body(j, carry, _ib=idx[b]):
                sl = pl.ds(j * _L, _L)
                _ib[sl] = _ib[sl] + off_v[sl]
                return carry

            lax.fori_loop(0, _CHUNK // _L, add_body, 0)
            gh[b] = pltpu.async_copy(table_hbm.at[idx[b]], rows[b], gsem[b])
            if c >= 1:
                pb = 1 - b
                gh[pb].wait()
                sh[pb] = pltpu.async_copy(
                    rows[pb],
                    out_hbm.at[pl.ds(base + (c - 1) * _CHUNK, _CHUNK)],
                    ssem[pb],
                )
        last = (_NCHUNK - 1) % 2
        gh[last].wait()
        sh[last] = pltpu.async_copy(
            rows[last],
            out_hbm.at[pl.ds(base + (_NCHUNK - 1) * _CHUNK, _CHUNK)],
            ssem[last],
        )
        sh[0].wait()
        sh[1].wait()

    return gather_kernel


_SC_TRANSPOSE = _make_sc_transpose()
_GATHER = _make_gather()
_OUTFMT = _make_outfmt()


def kernel(x, table):
    x_flat = x.reshape(_N)
    offs = jnp.asarray(_OFFS_TILED_NP)
    t4 = table.T.reshape(2, 8, _NUM_EMB)
    tail_flat = table[_TILES_FULL * 128:, :].reshape(_TAIL_COLS * _E)
    table_lin = _SC_TRANSPOSE(t4, tail_flat).reshape(_NUM_EMB, _E)
    rows = _GATHER(x_flat, offs, table_lin)
    out3 = _OUTFMT(rows.reshape(_N * _E))
    return out3.transpose(2, 0, 1)
